# Initial kernel scaffold; baseline (speedup 1.0000x reference)
import jax
import jax.numpy as jnp
from jax.experimental import pallas as pl


def kernel(emb_weight, input_values, dropout_mask_uniform):
    raise NotImplementedError("write your pallas kernel here")


# speedup vs baseline: 2.4249x; 2.4249x over previous
import functools

import jax
import jax.numpy as jnp
from jax import lax
from jax.experimental import pallas as pl
from jax.experimental.pallas import tpu as pltpu
from jax.experimental.pallas import tpu_sc as plsc

_DROPOUT = 0.5
_KEEP = 1.0 - _DROPOUT
_SCALE = 1.0 / _KEEP

_NC, _NS, _L = 2, 16, 16
_NW = _NC * _NS
_R = 128


@functools.partial(jax.jit, static_argnums=(3, 4, 5, 6))
def _run(emb_weight, idx, mask_flat, V, D, B, S):
    N = B * S
    per_w = N // _NW
    n_ch = per_w // _R
    mesh = plsc.VectorSubcoreMesh(core_axis_name="c", subcore_axis_name="s")

    @functools.partial(
        pl.kernel,
        mesh=mesh,
        out_type=jax.ShapeDtypeStruct((N, D), jnp.float32),
        scratch_types=[
            pltpu.VMEM((n_ch, _R), jnp.int32),
            pltpu.VMEM((_R,), jnp.int32),
            pltpu.VMEM((_R,), jnp.float32),
            pltpu.VMEM((_R,), jnp.float32),
            pltpu.VMEM((_R, D), jnp.float32),
            pltpu.SemaphoreType.DMA,
            pltpu.SemaphoreType.DMA,
        ],
    )
    def k(table_hbm, idx_hbm, mask_hbm, out_hbm,
          idx_v, midx_v, mval_v, scale_v, rows_v, sem_r, sem_m):
        wid = lax.axis_index("s") * _NC + lax.axis_index("c")
        pltpu.sync_copy(idx_hbm.at[wid], idx_v)
        base = wid * per_w

        def chunk(c, carry):
            row0 = base + c * _R
            row_cp = pltpu.async_copy(table_hbm.at[idx_v.at[c]], rows_v, sem_r)
            for j in range(_R // _L):
                pos = row0 + j * _L + lax.iota(jnp.int32, _L)
                b = lax.div(pos, jnp.full((_L,), S, jnp.int32))
                midx_v[pl.ds(j * _L, _L)] = b * V + idx_v[c, pl.ds(j * _L, _L)]
            pltpu.async_copy(mask_hbm.at[midx_v], mval_v, sem_m).wait()
            for j in range(_R // _L):
                mv = mval_v[pl.ds(j * _L, _L)]
                scale_v[pl.ds(j * _L, _L)] = jnp.where(mv < _KEEP, _SCALE, 0.0)
            row_cp.wait()

            def mul_g(g, carry2):
                s_vec = scale_v[pl.ds(g * _L, _L)]
                for i in range(_L):
                    r = g * _L + i
                    s = s_vec[i]
                    for d0 in range(D // _L):
                        rows_v[r, pl.ds(d0 * _L, _L)] = (
                            rows_v[r, pl.ds(d0 * _L, _L)] * s)
                return carry2

            lax.fori_loop(0, _R // _L, mul_g, 0)

            pltpu.sync_copy(rows_v, out_hbm.at[pl.ds(row0, _R)])
            return carry

        lax.fori_loop(0, n_ch, chunk, 0)

    return k(emb_weight, idx, mask_flat)


def kernel(emb_weight, input_values, dropout_mask_uniform):
    B, S = input_values.shape
    V, D = emb_weight.shape
    N = B * S
    idx = input_values.astype(jnp.int32).reshape(_NW, N // _NW // _R, _R)
    mask_flat = dropout_mask_uniform.reshape(-1)
    out = _run(emb_weight, idx, mask_flat, V, D, B, S)
    return out.reshape(B, S, D)
